# baseline (speedup 1.0000x reference)
import functools

import jax
import jax.numpy as jnp
from jax import lax
from jax.experimental import pallas as pl
from jax.experimental.pallas import tpu as pltpu
from jax.experimental.pallas import tpu_sc as plsc

N = 10000
E = 320000
D = 128
DH = D // 2
NC, NS = 2, 16
NW = NC * NS
CH = 125
EPT = E // NW
NCH1 = EPT // CH
EPS = E // NS
CH3 = 125
NCH3 = EPS // CH3
NPAD = 10240
STRIPE = NPAD // NS

_mesh = plsc.VectorSubcoreMesh(core_axis_name="c", subcore_axis_name="s")


def _make_deg_kernel(which):
    @functools.partial(
        pl.kernel,
        out_type=jax.ShapeDtypeStruct((NC, NPAD, 16), jnp.float32),
        mesh=_mesh,
        scratch_types=[
            pltpu.VMEM((NCH1, CH), jnp.int32),
            pltpu.VMEM((CH, 16), jnp.float32),
            pltpu.VMEM_SHARED((NPAD, 16), jnp.float32),
            pltpu.SemaphoreType.DMA,
        ],
        compiler_params=pltpu.CompilerParams(use_tc_tiling_on_sc=False),
        name=f"deg_hist_{which}",
    )
    def deg_kernel(edges_hbm, e0_hbm, z16_hbm, degp_hbm,
                   idx, ev0, hist_sh, sem0):
        c = lax.axis_index("c")
        s = lax.axis_index("s")
        w = c * NS + s

        pltpu.async_copy(z16_hbm, hist_sh.at[pl.ds(s * STRIPE, STRIPE)], sem0)
        pltpu.async_copy(e0_hbm, ev0, sem0)
        pltpu.async_copy(edges_hbm.at[which, w], idx, sem0)
        pltpu.make_async_copy(z16_hbm,
                              hist_sh.at[pl.ds(s * STRIPE, STRIPE)],
                              sem0).wait()
        pltpu.make_async_copy(e0_hbm, ev0, sem0).wait()
        pltpu.make_async_copy(edges_hbm.at[which, w], idx, sem0).wait()
        plsc.subcore_barrier()

        W = 4

        def scat(g):
            pltpu.async_copy(ev0, hist_sh.at[idx.at[g]], sem0, add=True)

        def wait_one(g):
            pltpu.make_async_copy(ev0, hist_sh.at[idx.at[g]], sem0).wait()

        for g in range(W):
            scat(g)

        def body(g, carry):
            scat(g)
            wait_one(g - W)
            return carry

        lax.fori_loop(W, NCH1, body, 0)
        for g in range(NCH1 - W, NCH1):
            wait_one(g)
        plsc.subcore_barrier()
        pltpu.sync_copy(hist_sh.at[pl.ds(s * STRIPE, STRIPE)],
                        degp_hbm.at[c, pl.ds(s * STRIPE, STRIPE)])

    return deg_kernel


_deg_out_kernel = _make_deg_kernel(0)
_deg_in_kernel = _make_deg_kernel(1)


@functools.partial(
    pl.kernel,
    out_type=jax.ShapeDtypeStruct((NC, NPAD, DH), jnp.float32),
    mesh=_mesh,
    scratch_types=[
        pltpu.VMEM((NCH3, CH3), jnp.int32),
        pltpu.VMEM((NCH3, CH3), jnp.int32),
        pltpu.VMEM((4, CH3, DH), jnp.float32),
        pltpu.VMEM_SHARED((NPAD, DH), jnp.float32),
        [pltpu.SemaphoreType.DMA] * 4,
        [pltpu.SemaphoreType.DMA] * 4,
    ],
    compiler_params=pltpu.CompilerParams(use_tc_tiling_on_sc=False),
)
def _agg_kernel(featsrc_hbm, edges_hbm, zrows_hbm, out_hbm,
                idx_s, idx_d, rows, h_sh, sem_g, sem_s):
    c = lax.axis_index("c")
    s = lax.axis_index("s")

    pltpu.async_copy(zrows_hbm, h_sh.at[pl.ds(s * STRIPE, STRIPE)], sem_g[0])
    pltpu.async_copy(edges_hbm.at[0, s], idx_s, sem_g[1])
    pltpu.async_copy(edges_hbm.at[1, s], idx_d, sem_g[2])
    pltpu.make_async_copy(zrows_hbm, h_sh.at[pl.ds(s * STRIPE, STRIPE)],
                          sem_g[0]).wait()
    pltpu.make_async_copy(edges_hbm.at[0, s], idx_s, sem_g[1]).wait()
    pltpu.make_async_copy(edges_hbm.at[1, s], idx_d, sem_g[2]).wait()
    plsc.subcore_barrier()

    fsrc = featsrc_hbm.at[c]

    def gather(g, b):
        pltpu.async_copy(fsrc.at[idx_s.at[g]], rows.at[b], sem_g[b])

    def wait_gather(g, b):
        pltpu.make_async_copy(fsrc.at[idx_s.at[g]], rows.at[b], sem_g[b]).wait()

    def scatter(g, b):
        pltpu.async_copy(rows.at[b], h_sh.at[idx_d.at[g]], sem_s[b], add=True)

    def wait_scatter(g, b):
        pltpu.make_async_copy(rows.at[b], h_sh.at[idx_d.at[g]],
                              sem_s[b]).wait()

    def step(g, b_main, b_pre, skip_ws=False, skip_gather=False):
        if not skip_ws:
            wait_scatter(g - 2, b_pre)
        if not skip_gather:
            gather(g + 2, b_pre)
        wait_gather(g, b_main)
        scatter(g, b_main)

    gather(0, 0)
    gather(1, 1)
    step(0, 0, 2, skip_ws=True)
    step(1, 1, 3, skip_ws=True)

    def body(gg, carry):
        g = 4 * gg + 2
        for k in range(4):
            step(g + k, (2 + k) % 4, k % 4)
        return carry

    lax.fori_loop(0, (NCH3 - 4) // 4, body, 0)
    step(NCH3 - 2, (NCH3 - 2) % 4, NCH3 % 4, skip_gather=True)
    step(NCH3 - 1, (NCH3 - 1) % 4, (NCH3 + 1) % 4, skip_gather=True)
    wait_scatter(NCH3 - 2, (NCH3 - 2) % 4)
    wait_scatter(NCH3 - 1, (NCH3 - 1) % 4)

    plsc.subcore_barrier()
    pltpu.sync_copy(h_sh.at[pl.ds(s * STRIPE, STRIPE)],
                    out_hbm.at[c, pl.ds(s * STRIPE, STRIPE)])


_RB = 2000


def _scale_body(feat_ref, deg_ref, out_ref):
    d = deg_ref[0, :, 0:1] + deg_ref[1, :, 0:1]
    norm = lax.rsqrt(jnp.maximum(d, 1.0) + 1.0)
    scaled = feat_ref[...] * norm
    out_ref[0] = scaled[:, :DH]
    out_ref[1] = scaled[:, DH:]


def _final_body(feat_ref, h_ref, deg_ref, out_ref):
    d = deg_ref[0, :, 0:1] + deg_ref[1, :, 0:1]
    norm = lax.rsqrt(jnp.maximum(d, 1.0) + 1.0)
    h = jnp.concatenate([h_ref[0], h_ref[1]], axis=1)
    out_ref[...] = (feat_ref[...] + h) * norm


def _scale(feat, degp):
    return pl.pallas_call(
        _scale_body,
        grid=(N // _RB,),
        in_specs=[
            pl.BlockSpec((_RB, D), lambda i: (i, 0)),
            pl.BlockSpec((NC, _RB, 16), lambda i: (0, i, 0)),
        ],
        out_specs=pl.BlockSpec((NC, _RB, DH), lambda i: (0, i, 0)),
        out_shape=jax.ShapeDtypeStruct((NC, N, DH), jnp.float32),
    )(feat, degp)


def _final(feat, h2, degp):
    return pl.pallas_call(
        _final_body,
        grid=(N // _RB,),
        in_specs=[
            pl.BlockSpec((_RB, D), lambda i: (i, 0)),
            pl.BlockSpec((NC, _RB, DH), lambda i: (0, i, 0)),
            pl.BlockSpec((NC, _RB, 16), lambda i: (0, i, 0)),
        ],
        out_specs=pl.BlockSpec((_RB, D), lambda i: (i, 0)),
        out_shape=jax.ShapeDtypeStruct((N, D), jnp.float32),
    )(feat, h2, degp)


def kernel(feat, edge_index):
    ei = edge_index.astype(jnp.int32)
    edges_k1 = ei.reshape(2, NW, NCH1, CH)
    edges_k3 = ei.reshape(2, NS, NCH3, CH3)

    e0 = jnp.zeros((CH, 16), jnp.float32).at[:, 0].set(1.0)
    z16 = jnp.zeros((STRIPE, 16), jnp.float32)
    zrows = jnp.zeros((STRIPE, DH), jnp.float32)

    dego = _deg_out_kernel(edges_k1, e0, z16)
    degi = _deg_in_kernel(edges_k1, e0, z16)
    feat_src = _scale(feat, dego)
    h2 = _agg_kernel(feat_src, edges_k3, zrows)
    return _final(feat, h2, degi)


# speedup vs baseline: 1.0011x; 1.0011x over previous
import functools

import jax
import jax.numpy as jnp
from jax import lax
from jax.experimental import pallas as pl
from jax.experimental.pallas import tpu as pltpu
from jax.experimental.pallas import tpu_sc as plsc

N = 10000
E = 320000
D = 128
DH = D // 2
NC, NS = 2, 16
NW = NC * NS
CH = 125
EPT = E // NW
NCH1 = EPT // CH
EPS = E // NS
CH3 = 125
NCH3 = EPS // CH3
NPAD = 10240
STRIPE = NPAD // NS

_mesh = plsc.VectorSubcoreMesh(core_axis_name="c", subcore_axis_name="s")


def _make_deg_kernel(which):
    @functools.partial(
        pl.kernel,
        out_type=jax.ShapeDtypeStruct((NC, NPAD, 16), jnp.float32),
        mesh=_mesh,
        scratch_types=[
            pltpu.VMEM((NCH1, CH), jnp.int32),
            pltpu.VMEM((CH, 16), jnp.float32),
            pltpu.VMEM_SHARED((NPAD, 16), jnp.float32),
            pltpu.SemaphoreType.DMA,
        ],
        compiler_params=pltpu.CompilerParams(use_tc_tiling_on_sc=False),
        name=f"deg_hist_{which}",
    )
    def deg_kernel(edges_hbm, e0_hbm, z16_hbm, degp_hbm,
                   idx, ev0, hist_sh, sem0):
        c = lax.axis_index("c")
        s = lax.axis_index("s")
        w = c * NS + s

        pltpu.sync_copy(z16_hbm, hist_sh.at[pl.ds(s * STRIPE, STRIPE)])
        pltpu.sync_copy(e0_hbm, ev0)
        pltpu.sync_copy(edges_hbm.at[which, w], idx)
        plsc.subcore_barrier()

        W = 4

        def scat(g):
            pltpu.async_copy(ev0, hist_sh.at[idx.at[g]], sem0, add=True)

        def wait_one(g):
            pltpu.make_async_copy(ev0, hist_sh.at[idx.at[g]], sem0).wait()

        for g in range(W):
            scat(g)

        def body(g, carry):
            scat(g)
            wait_one(g - W)
            return carry

        lax.fori_loop(W, NCH1, body, 0)
        for g in range(NCH1 - W, NCH1):
            wait_one(g)
        plsc.subcore_barrier()
        pltpu.sync_copy(hist_sh.at[pl.ds(s * STRIPE, STRIPE)],
                        degp_hbm.at[c, pl.ds(s * STRIPE, STRIPE)])

    return deg_kernel


_deg_out_kernel = _make_deg_kernel(0)
_deg_in_kernel = _make_deg_kernel(1)


@functools.partial(
    pl.kernel,
    out_type=jax.ShapeDtypeStruct((NC, NPAD, DH), jnp.float32),
    mesh=_mesh,
    scratch_types=[
        pltpu.VMEM((NCH3, CH3), jnp.int32),
        pltpu.VMEM((NCH3, CH3), jnp.int32),
        pltpu.VMEM((4, CH3, DH), jnp.float32),
        pltpu.VMEM_SHARED((NPAD, DH), jnp.float32),
        [pltpu.SemaphoreType.DMA] * 4,
        [pltpu.SemaphoreType.DMA] * 4,
    ],
    compiler_params=pltpu.CompilerParams(use_tc_tiling_on_sc=False),
)
def _agg_kernel(featsrc_hbm, edges_hbm, zrows_hbm, out_hbm,
                idx_s, idx_d, rows, h_sh, sem_g, sem_s):
    c = lax.axis_index("c")
    s = lax.axis_index("s")

    pltpu.sync_copy(zrows_hbm, h_sh.at[pl.ds(s * STRIPE, STRIPE)])
    pltpu.sync_copy(edges_hbm.at[0, s], idx_s)
    pltpu.sync_copy(edges_hbm.at[1, s], idx_d)
    plsc.subcore_barrier()

    fsrc = featsrc_hbm.at[c]

    def gather(g, b):
        pltpu.async_copy(fsrc.at[idx_s.at[g]], rows.at[b], sem_g[b])

    def wait_gather(g, b):
        pltpu.make_async_copy(fsrc.at[idx_s.at[g]], rows.at[b], sem_g[b]).wait()

    def scatter(g, b):
        pltpu.async_copy(rows.at[b], h_sh.at[idx_d.at[g]], sem_s[b], add=True)

    def wait_scatter(g, b):
        pltpu.make_async_copy(rows.at[b], h_sh.at[idx_d.at[g]],
                              sem_s[b]).wait()

    def step(g, b_main, b_pre, skip_ws=False, skip_gather=False):
        if not skip_ws:
            wait_scatter(g - 2, b_pre)
        if not skip_gather:
            gather(g + 2, b_pre)
        wait_gather(g, b_main)
        scatter(g, b_main)

    gather(0, 0)
    gather(1, 1)
    step(0, 0, 2, skip_ws=True)
    step(1, 1, 3, skip_ws=True)

    def body(gg, carry):
        g = 4 * gg + 2
        for k in range(4):
            step(g + k, (2 + k) % 4, k % 4)
        return carry

    lax.fori_loop(0, (NCH3 - 4) // 4, body, 0)
    step(NCH3 - 2, (NCH3 - 2) % 4, NCH3 % 4, skip_gather=True)
    step(NCH3 - 1, (NCH3 - 1) % 4, (NCH3 + 1) % 4, skip_gather=True)
    wait_scatter(NCH3 - 2, (NCH3 - 2) % 4)
    wait_scatter(NCH3 - 1, (NCH3 - 1) % 4)

    plsc.subcore_barrier()
    pltpu.sync_copy(h_sh.at[pl.ds(s * STRIPE, STRIPE)],
                    out_hbm.at[c, pl.ds(s * STRIPE, STRIPE)])


_RB = 2000


def _scale_body(feat_ref, deg_ref, out_ref):
    d = deg_ref[0, :, 0:1] + deg_ref[1, :, 0:1]
    norm = lax.rsqrt(jnp.maximum(d, 1.0) + 1.0)
    scaled = feat_ref[...] * norm
    out_ref[0] = scaled[:, :DH]
    out_ref[1] = scaled[:, DH:]


def _final_body(feat_ref, h_ref, deg_ref, out_ref):
    d = deg_ref[0, :, 0:1] + deg_ref[1, :, 0:1]
    norm = lax.rsqrt(jnp.maximum(d, 1.0) + 1.0)
    h = jnp.concatenate([h_ref[0], h_ref[1]], axis=1)
    out_ref[...] = (feat_ref[...] + h) * norm


def _scale(feat, degp):
    return pl.pallas_call(
        _scale_body,
        grid=(N // _RB,),
        in_specs=[
            pl.BlockSpec((_RB, D), lambda i: (i, 0)),
            pl.BlockSpec((NC, _RB, 16), lambda i: (0, i, 0)),
        ],
        out_specs=pl.BlockSpec((NC, _RB, DH), lambda i: (0, i, 0)),
        out_shape=jax.ShapeDtypeStruct((NC, N, DH), jnp.float32),
    )(feat, degp)


def _final(feat, h2, degp):
    return pl.pallas_call(
        _final_body,
        grid=(N // _RB,),
        in_specs=[
            pl.BlockSpec((_RB, D), lambda i: (i, 0)),
            pl.BlockSpec((NC, _RB, DH), lambda i: (0, i, 0)),
            pl.BlockSpec((NC, _RB, 16), lambda i: (0, i, 0)),
        ],
        out_specs=pl.BlockSpec((_RB, D), lambda i: (i, 0)),
        out_shape=jax.ShapeDtypeStruct((N, D), jnp.float32),
    )(feat, h2, degp)


def kernel(feat, edge_index):
    ei = edge_index.astype(jnp.int32)
    edges_k1 = ei.reshape(2, NW, NCH1, CH)
    edges_k3 = ei.reshape(2, NS, NCH3, CH3)

    e0 = jnp.zeros((CH, 16), jnp.float32).at[:, 0].set(1.0)
    z16 = jnp.zeros((STRIPE, 16), jnp.float32)
    zrows = jnp.zeros((STRIPE, DH), jnp.float32)

    dego = _deg_out_kernel(edges_k1, e0, z16)
    degi = _deg_in_kernel(edges_k1, e0, z16)
    feat_src = _scale(feat, dego)
    h2 = _agg_kernel(feat_src, edges_k3, zrows)
    return _final(feat, h2, degi)
